# Initial kernel scaffold; baseline (speedup 1.0000x reference)
import jax
import jax.numpy as jnp
from jax.experimental import pallas as pl


def kernel(input_len, table):
    raise NotImplementedError("write your pallas kernel here")


# speedup vs baseline: 3.4667x; 3.4667x over previous
import functools

import jax
import jax.numpy as jnp
from jax import lax
from jax.experimental import pallas as pl
from jax.experimental.pallas import tpu as pltpu
from jax.experimental.pallas import tpu_sc as plsc

D_MODEL = 1024
SEQ = 4096
BATCH = 16
LANES = 16

NUM_CORES = 2
NUM_SUBCORES = 16
NW = NUM_CORES * NUM_SUBCORES

CHUNK = 32
CHUNKS_PER_TILE = SEQ // (CHUNK * NW)
POS_PER_TILE = SEQ // NW
VPR = D_MODEL // LANES


_MESH = plsc.VectorSubcoreMesh(core_axis_name="c", subcore_axis_name="s")


@functools.partial(
    pl.kernel,
    mesh=_MESH,
    out_type=(
        jax.ShapeDtypeStruct((BATCH, SEQ, D_MODEL), jnp.float32),
        jax.ShapeDtypeStruct((BATCH * SEQ,), jnp.int32),
    ),
    scratch_types=[
        pltpu.VMEM((LANES,), jnp.int32),
        pltpu.VMEM((CHUNK, D_MODEL), jnp.float32),
        pltpu.VMEM((CHUNK, D_MODEL), jnp.float32),
        pltpu.VMEM((CHUNK, D_MODEL), jnp.float32),
        pltpu.VMEM((POS_PER_TILE,), jnp.int32),
    ],
)
def _pe_sc(len_hbm, tbl_hbm, zeros_hbm, emb_hbm, pos_hbm,
           len_v, tbl_v, zero_v, merge_v, pos_v):
    wid = lax.axis_index("s") * NUM_CORES + lax.axis_index("c")
    pltpu.sync_copy(len_hbm, len_v)
    pltpu.sync_copy(zeros_hbm, zero_v)
    len_vec = len_v[...]

    s0p = wid * POS_PER_TILE

    for b in range(BATCH):
        lb = len_vec[b]

        def pos_i(i, c, lb=lb):
            p = s0p + i * LANES + 1 + lax.iota(jnp.int32, LANES)
            pos_v[pl.ds(i * LANES, LANES)] = jnp.where(p <= lb, p, 0)
            return c

        lax.fori_loop(0, POS_PER_TILE // LANES, pos_i, 0)
        pltpu.sync_copy(pos_v, pos_hbm.at[pl.ds(b * SEQ + s0p, POS_PER_TILE)])

    def chunk_body(c, carry):
        s0 = (wid * CHUNKS_PER_TILE + c) * CHUNK
        pltpu.sync_copy(tbl_hbm.at[pl.ds(s0, CHUNK)], tbl_v)

        for b in range(BATCH):
            lb = len_vec[b]
            n = lb - s0

            @pl.when(n >= CHUNK)
            def _():
                pltpu.sync_copy(tbl_v, emb_hbm.at[b, pl.ds(s0, CHUNK)])

            @pl.when(n <= 0)
            def _():
                pltpu.sync_copy(zero_v, emb_hbm.at[b, pl.ds(s0, CHUNK)])

            @pl.when(jnp.logical_and(n > 0, n < CHUNK))
            def _(n=n, b=b, s0=s0):
                def row(r, rc):
                    keep = r < n

                    def col(j, cc):
                        v = tbl_v[r, pl.ds(j * LANES, LANES)]
                        merge_v[r, pl.ds(j * LANES, LANES)] = jnp.where(
                            keep, v, 0.0)
                        return cc

                    lax.fori_loop(0, VPR, col, 0)
                    return rc

                lax.fori_loop(0, CHUNK, row, 0)
                pltpu.sync_copy(merge_v, emb_hbm.at[b, pl.ds(s0, CHUNK)])

        return carry

    lax.fori_loop(0, CHUNKS_PER_TILE, chunk_body, 0)


def kernel(input_len, table):
    len32 = input_len.astype(jnp.int32)
    tbl = table[1:]
    zeros = jnp.zeros((CHUNK, D_MODEL), jnp.float32)
    emb, pos_flat = _pe_sc(len32, tbl, zeros)
    return emb, pos_flat.reshape(BATCH, SEQ)
